# Initial kernel scaffold; baseline (speedup 1.0000x reference)
import functools
import jax
import jax.numpy as jnp
from jax import lax
from jax.experimental import pallas as pl
from jax.experimental.pallas import tpu as pltpu
from jax.experimental.pallas import tpu_sc as plsc

_NUM_CAT = 100
_MOST = 10000
_CDIM = 32
_B = 16384

_NC = 2
_NS = 16
_NW = _NC * _NS
_BPW = _B // _NW
_NB = 16
_NCHUNK = _BPW // _NB
_CHUNK_IDX = _NB * _NUM_CAT
_NVREG = _CHUNK_IDX // 16


def _body(x_hbm, mask_hbm, e_hbm, out_hbm,
          idx_v, mask_v, shift_v, rows_v, out_v, gsem):
    wid = lax.axis_index("s") * _NC + lax.axis_index("c")
    base = wid * _BPW

    def shift_body(j, _):
        pos = jnp.full((16,), j * 16, jnp.int32) + lax.iota(jnp.int32, 16)
        cat = lax.rem(pos, jnp.full((16,), _NUM_CAT, jnp.int32))
        shift_v[pl.ds(j * 16, 16)] = cat * _MOST
        return 0

    lax.fori_loop(0, _NVREG, shift_body, 0)

    def chunk_body(k, _):
        b0 = base + k * _NB
        pltpu.sync_copy(x_hbm.at[pl.ds(b0 * _NUM_CAT, _CHUNK_IDX)], idx_v)
        pltpu.sync_copy(mask_hbm.at[pl.ds(b0 * _NUM_CAT, _CHUNK_IDX)], mask_v)

        def add_shift(j, _):
            s = pl.ds(j * 16, 16)
            idx_v[s] = idx_v[s] + shift_v[s]
            return 0

        lax.fori_loop(0, _NVREG, add_shift, 0)

        copies = [
            pltpu.async_copy(
                e_hbm.at[idx_v.at[pl.ds(i * _NUM_CAT, _NUM_CAT)]],
                rows_v.at[i], gsem)
            for i in range(_NB)
        ]
        for c in copies:
            c.wait()

        def batch_body(i, _):
            def cat_body(c, carry):
                a0, a1 = carry
                m = plsc.load_gather(
                    mask_v, [jnp.full((16,), i * _NUM_CAT + c, jnp.int32)])
                r0 = rows_v[i, c, pl.ds(0, 16)]
                r1 = rows_v[i, c, pl.ds(16, 16)]
                return a0 + m * r0, a1 + m * r1

            zero = jnp.zeros((16,), jnp.float32)
            a0, a1 = lax.fori_loop(0, _NUM_CAT, cat_body, (zero, zero))
            out_v[i, pl.ds(0, 16)] = a0
            out_v[i, pl.ds(16, 16)] = a1
            return 0

        lax.fori_loop(0, _NB, batch_body, 0)
        pltpu.sync_copy(out_v, out_hbm.at[pl.ds(b0, _NB)])
        return 0

    lax.fori_loop(0, _NCHUNK, chunk_body, 0)


def kernel(x, mask, E):
    mesh = plsc.VectorSubcoreMesh(core_axis_name="c", subcore_axis_name="s")
    run = functools.partial(
        pl.kernel,
        out_type=jax.ShapeDtypeStruct((_B, _CDIM), jnp.float32),
        mesh=mesh,
        scratch_types=[
            pltpu.VMEM((_CHUNK_IDX,), jnp.int32),
            pltpu.VMEM((_CHUNK_IDX,), jnp.float32),
            pltpu.VMEM((_CHUNK_IDX,), jnp.int32),
            pltpu.VMEM((_NB, _NUM_CAT, _CDIM), jnp.float32),
            pltpu.VMEM((_NB, _CDIM), jnp.float32),
            pltpu.SemaphoreType.DMA,
        ],
    )(_body)
    return run(x.reshape(-1), mask.reshape(-1), E)


# speedup vs baseline: 8.7308x; 8.7308x over previous
import functools
import jax
import jax.numpy as jnp
from jax import lax
from jax.experimental import pallas as pl
from jax.experimental.pallas import tpu as pltpu
from jax.experimental.pallas import tpu_sc as plsc

_NUM_CAT = 100
_MOST = 10000
_CDIM = 32
_B = 16384

_NC = 2
_NS = 16
_NW = _NC * _NS
_BPW = _B // _NW
_NB = 16
_NCHUNK = _BPW // _NB
_CHUNK_IDX = _NB * _NUM_CAT
_NVREG = _CHUNK_IDX // 16


def _body(x_hbm, mask_hbm, e_hbm, out_hbm,
          idx_v, mask_v, shift_v, rows_v, out_v, gsem):
    wid = lax.axis_index("s") * _NC + lax.axis_index("c")
    base = wid * _BPW

    def shift_body(j, _):
        pos = jnp.full((16,), j * 16, jnp.int32) + lax.iota(jnp.int32, 16)
        cat = lax.rem(pos, jnp.full((16,), _NUM_CAT, jnp.int32))
        shift_v[pl.ds(j * 16, 16)] = cat * _MOST
        return 0

    lax.fori_loop(0, _NVREG, shift_body, 0)

    def chunk_body(k, _):
        b0 = base + k * _NB
        pltpu.sync_copy(x_hbm.at[pl.ds(b0 * _NUM_CAT, _CHUNK_IDX)], idx_v)
        pltpu.sync_copy(mask_hbm.at[pl.ds(b0 * _NUM_CAT, _CHUNK_IDX)],
                        mask_v.at[pl.ds(0, _CHUNK_IDX)])

        def add_shift(j, _):
            s = pl.ds(j * 16, 16)
            idx_v[s] = idx_v[s] + shift_v[s]
            return 0

        lax.fori_loop(0, _NVREG, add_shift, 0)

        pltpu.async_copy(e_hbm.at[idx_v], rows_v, gsem).wait()

        def batch_body(i, _):
            ibase = i * _NUM_CAT
            a0 = jnp.zeros((16,), jnp.float32)
            a1 = jnp.zeros((16,), jnp.float32)
            for blk in range(7):
                m_vec = mask_v[pl.ds(ibase + blk * 16, 16)]
                for lane in range(16 if blk < 6 else _NUM_CAT - 96):
                    c = blk * 16 + lane
                    m = jnp.full((16,), m_vec[lane], jnp.float32)
                    a0 = a0 + m * rows_v[ibase + c, pl.ds(0, 16)]
                    a1 = a1 + m * rows_v[ibase + c, pl.ds(16, 16)]
            out_v[i, pl.ds(0, 16)] = a0
            out_v[i, pl.ds(16, 16)] = a1
            return 0

        lax.fori_loop(0, _NB, batch_body, 0)
        pltpu.sync_copy(out_v, out_hbm.at[pl.ds(b0, _NB)])
        return 0

    lax.fori_loop(0, _NCHUNK, chunk_body, 0)


def kernel(x, mask, E):
    mesh = plsc.VectorSubcoreMesh(core_axis_name="c", subcore_axis_name="s")
    run = functools.partial(
        pl.kernel,
        out_type=jax.ShapeDtypeStruct((_B, _CDIM), jnp.float32),
        mesh=mesh,
        compiler_params=pltpu.CompilerParams(use_tc_tiling_on_sc=False),
        scratch_types=[
            pltpu.VMEM((_CHUNK_IDX,), jnp.int32),
            pltpu.VMEM((_CHUNK_IDX + 16,), jnp.float32),
            pltpu.VMEM((_CHUNK_IDX,), jnp.int32),
            pltpu.VMEM((_CHUNK_IDX, _CDIM), jnp.float32),
            pltpu.VMEM((_NB, _CDIM), jnp.float32),
            pltpu.SemaphoreType.DMA,
        ],
    )(_body)
    return run(x.reshape(-1), mask.reshape(-1), E)
